# baseline (speedup 1.0000x reference)
import jax
import jax.numpy as jnp
from jax import lax
from jax.experimental import pallas as pl
from jax.experimental.pallas import tpu as pltpu
from jax.experimental.pallas import tpu_sc as plsc

N = 10000
E = 320000
HID = 128
OUT = 64
L_PE = 8
N_PERM = 8

NPAD = 10240
BPT = NPAD // 32
GK = 64

BNF = 1000

_mesh = plsc.VectorSubcoreMesh(core_axis_name="c", subcore_axis_name="s")


def _gather_body(tab_hbm, idx_hbm, out_hbm, idx_v, rows_v, sem):
    ci = lax.axis_index("c")
    sid = lax.axis_index("s")
    base = (ci * 16 + sid) * BPT

    def chunk(i, carry):
        off = base + i * GK
        pltpu.sync_copy(idx_hbm.at[pl.ds(off, GK)], idx_v)
        pltpu.async_copy(tab_hbm.at[idx_v], rows_v, sem).wait()
        pltpu.sync_copy(rows_v, out_hbm.at[pl.ds(off, GK)])
        return carry

    lax.fori_loop(0, BPT // GK, chunk, 0)


_gather_call = pl.kernel(
    _gather_body,
    mesh=_mesh,
    out_type=jax.ShapeDtypeStruct((NPAD, 128), jnp.float32),
    scratch_types=[
        pltpu.VMEM((GK,), jnp.int32),
        pltpu.VMEM((GK, 128), jnp.float32),
        pltpu.SemaphoreType.DMA,
    ],
)


def _gcn_mm_body(a_ref, d_ref, w_ref, b_ref, o_ref):
    a = a_ref[...] / jnp.maximum(d_ref[...], 1.0)
    o_ref[...] = jax.nn.relu(
        jnp.dot(a, w_ref[...], preferred_element_type=jnp.float32) + b_ref[...]
    )


def _gcn_mm(agg, deg, w, b):
    return pl.pallas_call(
        _gcn_mm_body,
        grid=(N // BNF,),
        in_specs=[
            pl.BlockSpec((BNF, HID), lambda i: (i, 0)),
            pl.BlockSpec((BNF, 1), lambda i: (i, 0)),
            pl.BlockSpec((HID, HID), lambda i: (0, 0)),
            pl.BlockSpec((1, HID), lambda i: (0, 0)),
        ],
        out_specs=pl.BlockSpec((BNF, HID), lambda i: (i, 0)),
        out_shape=jax.ShapeDtypeStruct((N, HID), jnp.float32),
    )(agg, deg.reshape(N, 1), w, b.reshape(1, HID))


def _mm_stats_body(a_ref, w_ref, b_ref, o_ref, s_ref, ss_ref):
    i = pl.program_id(0)
    y = jnp.dot(a_ref[...], w_ref[...], preferred_element_type=jnp.float32) + b_ref[...]
    o_ref[...] = y

    @pl.when(i == 0)
    def _init():
        s_ref[...] = jnp.zeros_like(s_ref)
        ss_ref[...] = jnp.zeros_like(ss_ref)

    s_ref[...] += jnp.sum(y, axis=0, keepdims=True)
    ss_ref[...] += jnp.sum(y * y, axis=0, keepdims=True)


def _mm_stats(a, w, b):
    n, k = a.shape
    m = w.shape[1]
    return pl.pallas_call(
        _mm_stats_body,
        grid=(n // BNF,),
        in_specs=[
            pl.BlockSpec((BNF, k), lambda i: (i, 0)),
            pl.BlockSpec((k, m), lambda i: (0, 0)),
            pl.BlockSpec((1, m), lambda i: (0, 0)),
        ],
        out_specs=[
            pl.BlockSpec((BNF, m), lambda i: (i, 0)),
            pl.BlockSpec((1, m), lambda i: (0, 0)),
            pl.BlockSpec((1, m), lambda i: (0, 0)),
        ],
        out_shape=[
            jax.ShapeDtypeStruct((n, m), jnp.float32),
            jax.ShapeDtypeStruct((1, m), jnp.float32),
            jax.ShapeDtypeStruct((1, m), jnp.float32),
        ],
    )(a, w, b.reshape(1, m))


def _bn_relu_mm_body(y_ref, sc_ref, sh_ref, w_ref, b_ref, o_ref):
    h = jax.nn.relu(y_ref[...] * sc_ref[...] + sh_ref[...])
    o_ref[...] = (
        jnp.dot(h, w_ref[...], preferred_element_type=jnp.float32) + b_ref[...]
    )


def _bn_relu_mm(y, scale, shift, w, b):
    n, k = y.shape
    m = w.shape[1]
    return pl.pallas_call(
        _bn_relu_mm_body,
        grid=(n // BNF,),
        in_specs=[
            pl.BlockSpec((BNF, k), lambda i: (i, 0)),
            pl.BlockSpec((1, k), lambda i: (0, 0)),
            pl.BlockSpec((1, k), lambda i: (0, 0)),
            pl.BlockSpec((k, m), lambda i: (0, 0)),
            pl.BlockSpec((1, m), lambda i: (0, 0)),
        ],
        out_specs=pl.BlockSpec((BNF, m), lambda i: (i, 0)),
        out_shape=jax.ShapeDtypeStruct((n, m), jnp.float32),
    )(y, scale.reshape(1, k), shift.reshape(1, k), w, b.reshape(1, m))


def _bn_affine(s, ss, n, g, beta):
    mu = s[0] / n
    var = ss[0] / n - mu * mu
    inv = g / jnp.sqrt(var + 1e-5)
    return inv, beta - mu * inv


def kernel(x, edge_index, idx, W1, b1, W2, b2, W3, b3, Wf1, bf1, gf1, betaf1,
           Wf2, bf2, Wp1, bp1, gp1, betap1, Wp2, bp2, perm_table):
    n = x.shape[0]
    src, dst = edge_index[0], edge_index[1]

    x_aug = jnp.concatenate([x, jnp.ones((n, 1), jnp.float32)], axis=1)
    agg_aug = jax.ops.segment_sum(x_aug[src], dst, num_segments=n)
    agg, deg = agg_aug[:, :HID], agg_aug[:, HID]
    h = _gcn_mm(agg, deg, W1, b1)
    for (W, b) in ((W2, b2), (W3, b3)):
        agg = jax.ops.segment_sum(h[src], dst, num_segments=n)
        h = _gcn_mm(agg, deg, W, b)
    xs = h

    y1, s1, ss1 = _mm_stats(xs, Wf1, bf1)
    sc1, sh1 = _bn_affine(s1, ss1, n, gf1, betaf1)
    regions = _bn_relu_mm(y1, sc1, sh1, Wf2, bf2)

    pt128 = jnp.pad(perm_table, ((0, 0), (0, 128 - N_PERM)))
    idx_pad = jnp.pad(idx, (0, NPAD - N))
    c = _gather_call(pt128, idx_pad)[:N, :N_PERM]

    coors = [c]
    for _ in range(L_PE - 1):
        m = jax.ops.segment_max(c[src], dst, num_segments=n)
        c = jnp.maximum(c, m)
        coors.append(c)
    trans = jnp.stack(coors, axis=0).transpose(1, 2, 0).reshape(n, N_PERM * L_PE)

    y2, s2, ss2 = _mm_stats(trans, Wp1, bp1)
    sc2, sh2 = _bn_affine(s2, ss2, n, gp1, betap1)
    pe = _bn_relu_mm(y2, sc2, sh2, Wp2, bp2)
    return (regions, pe)
